# baseline (speedup 1.0000x reference)
import functools

import jax
import jax.numpy as jnp
from jax.experimental import pallas as pl
from jax.experimental.pallas import tpu as pltpu

_NUM_LEVELS = 4
_BETA = 0.001


def _fused(nt, x_ref, q_ref, w_ref,
           eW0, eb0, eW1, eb1, eW2, eb2, eW3, eb3,
           dW0, db0, dW1, db1, dW2, db2, dW3, db3,
           cb_ref, cb3_ref,
           out_ref, rq_ref, idx_ref, xq_ref, qd_ref,
           cb2_ref):
    step = pl.program_id(0)

    @pl.when(step == 0)
    def _():
        cb2_ref[...] = jnp.sum(cb_ref[...] * cb_ref[...], axis=2)
    t = x_ref.shape[0]
    f32 = jnp.float32

    h = jnp.concatenate([x_ref[...], q_ref[...]], axis=0)
    enc = [(eW0, eb0), (eW1, eb1), (eW2, eb2), (eW3, eb3)]
    for i, (w, b) in enumerate(enc):
        h = jnp.dot(h, w[...], preferred_element_type=f32) + b[...]
        if i < len(enc) - 1:
            h = jax.nn.relu(h)
    x_e = h[:t]
    q_enc = h[t:]

    csize = cb_ref.shape[1]
    residual = x_e
    x_q = jnp.zeros_like(x_e)
    sq_sum = f32(0.0)
    idx_cols = []
    lane_iota = jax.lax.broadcasted_iota(jnp.int32, (t, csize), 1)
    dcode = x_e.shape[1]
    for l in range(_NUM_LEVELS):
        cb = cb_ref[l]
        cross = jax.lax.dot_general(residual, cb, (((1,), (1,)), ((), ())),
                                    preferred_element_type=f32)
        r2 = jnp.sum(residual * residual, axis=1, keepdims=True)
        cb2 = cb2_ref[l]
        d = r2 - 2.0 * cross + cb2[None, :]
        idx = jnp.argmin(d, axis=1)
        onehot = (lane_iota == idx[:, None]).astype(jnp.bfloat16)
        q3 = jnp.dot(onehot, cb3_ref[l], preferred_element_type=f32)
        q = (q3[:, :dcode] + q3[:, dcode:2 * dcode]) + q3[:, 2 * dcode:]
        diff = q - residual
        sq_sum = sq_sum + jnp.sum(diff * diff)
        x_q = x_q + q
        residual = residual - q
        idx_cols.append(idx[:, None])
    idx_ref[...] = jnp.concatenate(idx_cols, axis=1).astype(jnp.int32)
    xq_ref[...] = x_q

    h2 = x_q
    dec = [(dW0, db0), (dW1, db1), (dW2, db2), (dW3, db3)]
    for i, (w, b) in enumerate(dec):
        h2 = jnp.dot(h2, w[...], preferred_element_type=f32) + b[...]
        if i < len(dec) - 1:
            h2 = jax.nn.relu(h2)
    out_ref[...] = h2

    dots = jnp.sum(x_e * q_enc, axis=1, keepdims=True)
    nx = jnp.sqrt(jnp.sum(x_e * x_e, axis=1, keepdims=True))
    nq = jnp.sqrt(jnp.sum(q_enc * q_enc, axis=1, keepdims=True))
    cos = w_ref[...] * (dots / (nx * nq + 1e-8))
    cos_sum = jnp.sum(cos)

    total_rows = f32(t) * f32(nt)
    prev_rq = jnp.where(step == 0, jnp.zeros((1, 1), f32), rq_ref[...])
    new_rq = prev_rq + sq_sum
    rq_scale = (1.0 + _BETA) / (total_rows * f32(x_e.shape[1]))
    rq_ref[...] = jnp.where(step == nt - 1, new_rq * rq_scale, new_rq)

    prev_qd = jnp.where(step == 0, jnp.zeros((1, 1), f32), qd_ref[...])
    new_qd = prev_qd + cos_sum
    qd_ref[...] = jnp.where(step == nt - 1, 1.0 - new_qd / total_rows, new_qd)


def kernel(x, q_embs, labels, qd_align_w,
           enc_W0, enc_b0, enc_W1, enc_b1, enc_W2, enc_b2, enc_W3, enc_b3,
           dec_W0, dec_b0, dec_W1, dec_b1, dec_W2, dec_b2, dec_W3, dec_b3,
           codebooks):
    del labels
    b, d_in = x.shape
    d_code = codebooks.shape[-1]
    tile = 1024
    nt = b // tile

    f32 = jnp.float32
    w2d = qd_align_w.reshape(b, 1).astype(f32)
    cb_hi = codebooks.astype(jnp.bfloat16)
    cb_mid = (codebooks - cb_hi.astype(f32)).astype(jnp.bfloat16)
    cb_lo = (codebooks - cb_hi.astype(f32) - cb_mid.astype(f32)).astype(jnp.bfloat16)
    cb3 = jnp.concatenate([cb_hi, cb_mid, cb_lo], axis=2)
    biases = [bb.reshape(1, -1) for bb in
              (enc_b0, enc_b1, enc_b2, enc_b3, dec_b0, dec_b1, dec_b2, dec_b3)]
    eb0, eb1, eb2, eb3, db0, db1, db2, db3 = biases

    def tile_spec(cols):
        return pl.BlockSpec((tile, cols), lambda i: (i, 0))

    def whole(a):
        return pl.BlockSpec(a.shape, lambda i: (0,) * a.ndim)

    in_specs = [
        tile_spec(d_in),
        tile_spec(d_in),
        tile_spec(1),
        whole(enc_W0), whole(eb0), whole(enc_W1), whole(eb1),
        whole(enc_W2), whole(eb2), whole(enc_W3), whole(eb3),
        whole(dec_W0), whole(db0), whole(dec_W1), whole(db1),
        whole(dec_W2), whole(db2), whole(dec_W3), whole(db3),
        whole(codebooks), whole(cb3),
    ]
    scalar_spec = pl.BlockSpec((1, 1), lambda i: (0, 0))
    out_specs = [
        tile_spec(d_in),
        scalar_spec,
        tile_spec(_NUM_LEVELS),
        tile_spec(d_code),
        scalar_spec,
    ]
    out_shapes = [
        jax.ShapeDtypeStruct((b, d_in), f32),
        jax.ShapeDtypeStruct((1, 1), f32),
        jax.ShapeDtypeStruct((b, _NUM_LEVELS), jnp.int32),
        jax.ShapeDtypeStruct((b, d_code), f32),
        jax.ShapeDtypeStruct((1, 1), f32),
    ]

    out, rq, indices, x_q, qd = pl.pallas_call(
        functools.partial(_fused, nt),
        grid=(nt,),
        in_specs=in_specs,
        out_specs=out_specs,
        out_shape=out_shapes,
        scratch_shapes=[pltpu.VMEM(codebooks.shape[:2], f32)],
    )(x, q_embs, w2d,
      enc_W0, eb0, enc_W1, eb1, enc_W2, eb2, enc_W3, eb3,
      dec_W0, db0, dec_W1, db1, dec_W2, db2, dec_W3, db3,
      codebooks, cb3)

    return (out, rq[0, 0], indices, x_q, qd[0, 0])


# speedup vs baseline: 1.0826x; 1.0826x over previous
import functools

import jax
import jax.numpy as jnp
from jax.experimental import pallas as pl
from jax.experimental.pallas import tpu as pltpu

_NUM_LEVELS = 4
_BETA = 0.001


def _fused(nt, x_ref, q_ref, w_ref,
           eW0, eb0, eW1, eb1, eW2, eb2, eW3, eb3,
           dW0, db0, dW1, db1, dW2, db2, dW3, db3,
           cb_ref, cb3_ref, cb2_ref,
           out_ref, rq_ref, idx_ref, xq_ref, qd_ref):
    step = pl.program_id(0)
    t = x_ref.shape[0]
    f32 = jnp.float32

    h = jnp.concatenate([x_ref[...], q_ref[...]], axis=0)
    enc = [(eW0, eb0), (eW1, eb1), (eW2, eb2), (eW3, eb3)]
    bf16 = jnp.bfloat16

    def dot2(a, w):
        return jnp.dot(a.astype(bf16), w.astype(bf16),
                       preferred_element_type=f32)

    for i, (w, b) in enumerate(enc):
        h = dot2(h, w[...]) + b[...]
        if i < len(enc) - 1:
            h = jax.nn.relu(h)
    x_e = h[:t]
    q_enc = h[t:]

    csize = cb_ref.shape[1]
    residual = x_e
    x_q = jnp.zeros_like(x_e)
    sq_sum = f32(0.0)
    idx_cols = []
    lane_iota = jax.lax.broadcasted_iota(jnp.int32, (t, csize), 1)
    dcode = x_e.shape[1]
    for l in range(_NUM_LEVELS):
        cb = cb_ref[l]
        dn = (((1,), (1,)), ((), ()))
        cross = jax.lax.dot_general(residual.astype(bf16), cb3_ref[l][:, :dcode],
                                    dn, preferred_element_type=f32)
        r2 = jnp.sum(residual * residual, axis=1, keepdims=True)
        cb2 = cb2_ref[l]
        d = (r2 - 2.0 * cross) + cb2[None, :]
        dmin = jnp.min(d, axis=1, keepdims=True)
        idx = jnp.min(jnp.where(d == dmin, lane_iota, csize), axis=1)
        onehot = (lane_iota == idx[:, None]).astype(jnp.bfloat16)
        q3 = jnp.dot(onehot, cb3_ref[l], preferred_element_type=f32)
        q = (q3[:, :dcode] + q3[:, dcode:2 * dcode]) + q3[:, 2 * dcode:]
        diff = q - residual
        sq_sum = sq_sum + jnp.sum(diff * diff)
        q_st = residual + diff
        x_q = x_q + q_st
        residual = residual - q_st
        idx_cols.append(idx[:, None])
    idx_ref[...] = jnp.concatenate(idx_cols, axis=1).astype(jnp.int32)
    xq_ref[...] = x_q

    h2 = x_q
    dec = [(dW0, db0), (dW1, db1), (dW2, db2), (dW3, db3)]
    for i, (w, b) in enumerate(dec):
        h2 = dot2(h2, w[...]) + b[...]
        if i < len(dec) - 1:
            h2 = jax.nn.relu(h2)
    out_ref[...] = h2

    dots = jnp.sum(x_e * q_enc, axis=1, keepdims=True)
    nx = jnp.sqrt(jnp.sum(x_e * x_e, axis=1, keepdims=True))
    nq = jnp.sqrt(jnp.sum(q_enc * q_enc, axis=1, keepdims=True))
    cos = w_ref[...] * (dots / (nx * nq + 1e-8))
    cos_sum = jnp.sum(cos)

    total_rows = f32(t) * f32(nt)
    prev_rq = jnp.where(step == 0, jnp.zeros((1, 1), f32), rq_ref[...])
    new_rq = prev_rq + sq_sum
    rq_scale = (1.0 + _BETA) / (total_rows * f32(x_e.shape[1]))
    rq_ref[...] = jnp.where(step == nt - 1, new_rq * rq_scale, new_rq)

    prev_qd = jnp.where(step == 0, jnp.zeros((1, 1), f32), qd_ref[...])
    new_qd = prev_qd + cos_sum
    qd_ref[...] = jnp.where(step == nt - 1, 1.0 - new_qd / total_rows, new_qd)


def kernel(x, q_embs, labels, qd_align_w,
           enc_W0, enc_b0, enc_W1, enc_b1, enc_W2, enc_b2, enc_W3, enc_b3,
           dec_W0, dec_b0, dec_W1, dec_b1, dec_W2, dec_b2, dec_W3, dec_b3,
           codebooks):
    del labels
    b, d_in = x.shape
    d_code = codebooks.shape[-1]
    tile = 512
    nt = b // tile

    f32 = jnp.float32
    w2d = qd_align_w.reshape(b, 1).astype(f32)
    cb_hi = codebooks.astype(jnp.bfloat16)
    cb_mid = (codebooks - cb_hi.astype(f32)).astype(jnp.bfloat16)
    cb_lo = (codebooks - cb_hi.astype(f32) - cb_mid.astype(f32)).astype(jnp.bfloat16)
    cb3 = jnp.concatenate([cb_hi, cb_mid, cb_lo], axis=2)
    cb2_all = jnp.stack([jnp.sum(codebooks[i] ** 2, axis=1)
                         for i in range(_NUM_LEVELS)])
    biases = [bb.reshape(1, -1) for bb in
              (enc_b0, enc_b1, enc_b2, enc_b3, dec_b0, dec_b1, dec_b2, dec_b3)]
    eb0, eb1, eb2, eb3, db0, db1, db2, db3 = biases

    def tile_spec(cols):
        return pl.BlockSpec((tile, cols), lambda i: (i, 0))

    def whole(a):
        return pl.BlockSpec(a.shape, lambda i: (0,) * a.ndim)

    in_specs = [
        tile_spec(d_in),
        tile_spec(d_in),
        tile_spec(1),
        whole(enc_W0), whole(eb0), whole(enc_W1), whole(eb1),
        whole(enc_W2), whole(eb2), whole(enc_W3), whole(eb3),
        whole(dec_W0), whole(db0), whole(dec_W1), whole(db1),
        whole(dec_W2), whole(db2), whole(dec_W3), whole(db3),
        whole(codebooks), whole(cb3), whole(cb2_all),
    ]
    scalar_spec = pl.BlockSpec((1, 1), lambda i: (0, 0))
    out_specs = [
        tile_spec(d_in),
        scalar_spec,
        tile_spec(_NUM_LEVELS),
        tile_spec(d_code),
        scalar_spec,
    ]
    out_shapes = [
        jax.ShapeDtypeStruct((b, d_in), f32),
        jax.ShapeDtypeStruct((1, 1), f32),
        jax.ShapeDtypeStruct((b, _NUM_LEVELS), jnp.int32),
        jax.ShapeDtypeStruct((b, d_code), f32),
        jax.ShapeDtypeStruct((1, 1), f32),
    ]

    out, rq, indices, x_q, qd = pl.pallas_call(
        functools.partial(_fused, nt),
        grid=(nt,),
        in_specs=in_specs,
        out_specs=out_specs,
        out_shape=out_shapes,
    )(x, q_embs, w2d,
      enc_W0, eb0, enc_W1, eb1, enc_W2, eb2, enc_W3, eb3,
      dec_W0, db0, dec_W1, db1, dec_W2, db2, dec_W3, db3,
      codebooks, cb3, cb2_all)

    return (out, rq[0, 0], indices, x_q, qd[0, 0])
